# baseline (speedup 1.0000x reference)
import jax
import jax.numpy as jnp
from jax.experimental import pallas as pl
from jax.experimental.pallas import tpu as pltpu

LEAKY_SLOPE = 0.01
BN_EPS = 1e-5
IPB = 2


def _prep_w(w_oihw):
    o, i, kh, kw = w_oihw.shape
    w = jnp.transpose(w_oihw, (2, 3, 1, 0)).reshape(kh, kw * i, o)
    return w.astype(jnp.bfloat16)


def _prep_w_wide(w_oihw):
    o, i, kh, kw = w_oihw.shape
    w = jnp.transpose(w_oihw, (2, 3, 1, 0)).reshape(kh, kw * i, o)
    w = jnp.concatenate([w[0], w[1], w[2], jnp.zeros((kw * i, o), w.dtype)],
                        axis=1)
    return w.astype(jnp.bfloat16)


def _lrelu(z):
    return jnp.where(z >= 0, z, LEAKY_SLOPE * z)


def _conv3x3_s1(a, w_ref):
    b, H, W, C = a.shape
    co = w_ref.shape[-1] // 4
    ab = a.astype(jnp.bfloat16)
    ap = jnp.pad(ab, ((0, 0), (1, 1), (1, 1), (0, 0)))
    xc = jnp.concatenate(
        [ap[:, :, 0:W, :], ap[:, :, 1:W + 1, :], ap[:, :, 2:W + 2, :]],
        axis=3)
    yw = jnp.dot(xc.reshape(b * (H + 2) * W, 3 * C), w_ref[...],
                 preferred_element_type=jnp.float32)
    yw = yw.reshape(b, H + 2, W, 4 * co)
    y = (yw[:, 0:H, :, 0:co] + yw[:, 1:H + 1, :, co:2 * co]
         + yw[:, 2:H + 2, :, 2 * co:3 * co])
    return y.reshape(b * H * W, co)


def _emit(y, y_ref, s_ref, q_ref):
    y_ref[...] = y.reshape(y_ref.shape).astype(y_ref.dtype)
    yr = y.reshape(-1, 8, y.shape[-1])
    s_ref[0] = jnp.sum(yr, axis=0)
    q_ref[0] = jnp.sum(yr * yr, axis=0)


def _coeffs_in(s_ref, q_ref, g_ref, b_ref, count):
    c = s_ref.shape[-1]
    s = jnp.sum(s_ref[...].reshape(-1, c), axis=0)
    q = jnp.sum(q_ref[...].reshape(-1, c), axis=0)
    mean = s / count
    var = jnp.maximum(q / count - mean * mean, 0.0)
    scale = g_ref[0] * jax.lax.rsqrt(var + BN_EPS)
    shift = b_ref[0] - mean * scale
    return scale, shift


def _k_first(x_ref, w_ref, y_ref, s_ref, q_ref):
    _emit(_conv3x3_s1(x_ref[...], w_ref), y_ref, s_ref, q_ref)


def _mk_mid(count):
    def body(yp_ref, w_ref, s_ref, q_ref, g_ref, b_ref,
             y_ref, so_ref, qo_ref):
        scale, shift = _coeffs_in(s_ref, q_ref, g_ref, b_ref, count)
        a = _lrelu(yp_ref[...] * scale + shift)
        _emit(_conv3x3_s1(a, w_ref), y_ref, so_ref, qo_ref)
    return body


def _mk_mid_res(count):
    def body(yp_ref, r_ref, w_ref, s_ref, q_ref, g_ref, b_ref,
             y_ref, so_ref, qo_ref):
        scale, shift = _coeffs_in(s_ref, q_ref, g_ref, b_ref, count)
        a = r_ref[...] + _lrelu(yp_ref[...] * scale + shift)
        _emit(_conv3x3_s1(a, w_ref), y_ref, so_ref, qo_ref)
    return body


def _mk_down(count):
    def body(y4_ref, y2_ref, x0_ref, w02_ref, w1_ref,
             s2_ref, q2_ref, g2_ref, b2_ref,
             s4_ref, q4_ref, g4_ref, b4_ref,
             y_ref, so_ref, qo_ref):
        sc2, sh2 = _coeffs_in(s2_ref, q2_ref, g2_ref, b2_ref, count)
        sc4, sh4 = _coeffs_in(s4_ref, q4_ref, g4_ref, b4_ref, count)
        a = (x0_ref[...] + _lrelu(y2_ref[...] * sc2 + sh2)
             + _lrelu(y4_ref[...] * sc4 + sh4))
        b, H, W, C = a.shape
        Ho, Wo = H // 2, W // 2
        cd = y_ref.shape[-1]
        ab = a.astype(jnp.bfloat16)
        ap = jnp.pad(ab, ((0, 0), (1, 1), (1, 1), (0, 0)))
        xc = jnp.concatenate(
            [ap[:, :, 0:W, :], ap[:, :, 1:W + 1, :], ap[:, :, 2:W + 2, :]],
            axis=3)
        hp = (H + 2) // 2
        xs = xc.reshape(b, hp, 2, W, 3 * C)
        xe, xo = xs[:, :, 0], xs[:, :, 1]
        ye = jnp.dot(xe.reshape(b * hp * W, 3 * C), w02_ref[...],
                     preferred_element_type=jnp.float32)
        ye = ye.reshape(b, hp, W, 2 * cd)
        yo = jnp.dot(xo[:, 0:Ho].reshape(b * Ho * W, 3 * C), w1_ref[...],
                     preferred_element_type=jnp.float32)
        acc = (ye[:, 0:Ho, :, 0:cd] + yo.reshape(b, Ho, W, cd)
               + ye[:, 1:Ho + 1, :, cd:2 * cd])
        y = acc.reshape(b, Ho, Wo, 2, cd)[:, :, :, 0, :]
        _emit(y.reshape(b * Ho * Wo, cd), y_ref, so_ref, qo_ref)
    return body


def _mk_bn(count):
    def body(y_ref, s_ref, q_ref, g_ref, b_ref, o_ref):
        scale, shift = _coeffs_in(s_ref, q_ref, g_ref, b_ref, count)
        o_ref[...] = _lrelu(y_ref[...] * scale + shift)
    return body


def _img_spec(h, w, c):
    return pl.BlockSpec((IPB, h, w, c), lambda i: (i, 0, 0, 0))


def _full3_spec(shape):
    return pl.BlockSpec(shape, lambda i: (0, 0, 0))


def _row_spec(c):
    return pl.BlockSpec((1, c), lambda i: (0, 0))


_CP = pltpu.CompilerParams(dimension_semantics=("parallel",),
                           vmem_limit_bytes=100 * 1024 * 1024)


def _stat_in_specs(g, c):
    return [_full3_spec((g, 8, c)), _full3_spec((g, 8, c)),
            _row_spec(c), _row_spec(c)]


def _conv_call(body, ins, in_specs, n, ho, wo, cout, n_extra=0):
    g = n // IPB
    out_shape = ((jax.ShapeDtypeStruct((n, ho, wo, cout), jnp.float32),)
                 * (1 + n_extra)
                 + (jax.ShapeDtypeStruct((g, 8, cout), jnp.float32),
                    jax.ShapeDtypeStruct((g, 8, cout), jnp.float32)))
    out_specs = ((_img_spec(ho, wo, cout),) * (1 + n_extra)
                 + (pl.BlockSpec((1, 8, cout), lambda i: (i, 0, 0)),
                    pl.BlockSpec((1, 8, cout), lambda i: (i, 0, 0))))
    return pl.pallas_call(
        body, out_shape=out_shape, grid=(g,), in_specs=in_specs,
        out_specs=out_specs, compiler_params=_CP)(*ins)


def kernel(x, res_w_0, res_b_0, res_g_0, res_be_0,
           res_w_1, res_b_1, res_g_1, res_be_1,
           down_w, down_b, down_g, down_be):
    n, c, h, w = x.shape
    cd = down_w.shape[0]
    ho, wo = h // 2, w // 2
    g = n // IPB
    m1 = n * h * w
    m5 = n * ho * wo

    x0 = jnp.transpose(x, (0, 2, 3, 1))
    w0 = _prep_w_wide(res_w_0)
    w1 = _prep_w_wide(res_w_1)
    wdp = _prep_w(down_w)
    wd02 = jnp.concatenate([wdp[0], wdp[2]], axis=1)
    wd1 = wdp[1]
    g0, be0 = res_g_0.reshape(1, c), res_be_0.reshape(1, c)
    g1, be1 = res_g_1.reshape(1, c), res_be_1.reshape(1, c)
    gd, bed = down_g.reshape(1, cd), down_be.reshape(1, cd)

    img = _img_spec(h, w, c)
    wsp = pl.BlockSpec((3 * c, 4 * c), lambda i: (0, 0))
    stat = _stat_in_specs(g, c)

    y1, s1, q1 = _conv_call(_k_first, (x0, w0), [img, wsp], n, h, w, c)
    y2, s2, q2 = _conv_call(_mk_mid(m1), (y1, w0, s1, q1, g0, be0),
                            [img, wsp] + stat, n, h, w, c)
    y3, s3, q3 = _conv_call(_mk_mid_res(m1), (y2, x0, w1, s2, q2, g0, be0),
                            [img, img, wsp] + stat, n, h, w, c)
    y4, s4, q4 = _conv_call(_mk_mid(m1), (y3, w1, s3, q3, g1, be1),
                            [img, wsp] + stat, n, h, w, c)
    y5, s5, q5 = _conv_call(
        _mk_down(m1),
        (y4, y2, x0, wd02, wd1, s2, q2, g0, be0, s4, q4, g1, be1),
        [img, img, img,
         pl.BlockSpec((3 * c, 2 * cd), lambda i: (0, 0)),
         pl.BlockSpec((3 * c, cd), lambda i: (0, 0))] + stat + stat,
        n, ho, wo, cd)

    out = pl.pallas_call(
        _mk_bn(m5),
        out_shape=jax.ShapeDtypeStruct((n, ho, wo, cd), jnp.float32),
        grid=(g,),
        in_specs=[_img_spec(ho, wo, cd)] + _stat_in_specs(g, cd),
        out_specs=_img_spec(ho, wo, cd),
        compiler_params=_CP)(y5, s5, q5, gd, bed)

    return jnp.transpose(out, (0, 3, 1, 2))


# speedup vs baseline: 1.0149x; 1.0149x over previous
import jax
import jax.numpy as jnp
from jax.experimental import pallas as pl
from jax.experimental.pallas import tpu as pltpu

LEAKY_SLOPE = 0.01
BN_EPS = 1e-5
IPB = 2


def _prep_w(w_oihw):
    o, i, kh, kw = w_oihw.shape
    w = jnp.transpose(w_oihw, (2, 3, 1, 0)).reshape(kh, kw * i, o)
    return w.astype(jnp.bfloat16)


def _prep_w_wide(w_oihw):
    o, i, kh, kw = w_oihw.shape
    w = jnp.transpose(w_oihw, (2, 3, 1, 0)).reshape(kh, kw * i, o)
    w = jnp.concatenate([w[0], w[1], w[2], jnp.zeros((kw * i, o), w.dtype)],
                        axis=1)
    return w.astype(jnp.bfloat16)


def _lrelu(z):
    return jnp.maximum(z, LEAKY_SLOPE * z)


def _conv3x3_s1(a, w_ref):
    b, H, W, C = a.shape
    co = w_ref.shape[-1] // 4
    ab = a.astype(jnp.bfloat16)
    ap = jnp.pad(ab, ((0, 0), (1, 1), (1, 1), (0, 0)))
    xc = jnp.concatenate(
        [ap[:, :, 0:W, :], ap[:, :, 1:W + 1, :], ap[:, :, 2:W + 2, :]],
        axis=3)
    yw = jnp.dot(xc.reshape(b * (H + 2) * W, 3 * C), w_ref[...],
                 preferred_element_type=jnp.float32)
    yw = yw.reshape(b, H + 2, W, 4 * co)
    y = (yw[:, 0:H, :, 0:co] + yw[:, 1:H + 1, :, co:2 * co]
         + yw[:, 2:H + 2, :, 2 * co:3 * co])
    return y.reshape(b * H * W, co)


def _emit(y, y_ref, s_ref, q_ref):
    y_ref[...] = y.reshape(y_ref.shape).astype(y_ref.dtype)
    yr = y.reshape(-1, 8, y.shape[-1])
    s_ref[0] = jnp.sum(yr, axis=0)
    q_ref[0] = jnp.sum(yr * yr, axis=0)


def _coeffs_in(s_ref, q_ref, g_ref, b_ref, count):
    c = s_ref.shape[-1]
    s = jnp.sum(s_ref[...].reshape(-1, c), axis=0)
    q = jnp.sum(q_ref[...].reshape(-1, c), axis=0)
    mean = s / count
    var = jnp.maximum(q / count - mean * mean, 0.0)
    scale = g_ref[0] * jax.lax.rsqrt(var + BN_EPS)
    shift = b_ref[0] - mean * scale
    return scale, shift


def _k_first(x_ref, w_ref, y_ref, s_ref, q_ref):
    _emit(_conv3x3_s1(x_ref[...], w_ref), y_ref, s_ref, q_ref)


def _mk_mid(count):
    def body(yp_ref, w_ref, s_ref, q_ref, g_ref, b_ref,
             y_ref, so_ref, qo_ref):
        scale, shift = _coeffs_in(s_ref, q_ref, g_ref, b_ref, count)
        a = _lrelu(yp_ref[...] * scale + shift)
        _emit(_conv3x3_s1(a, w_ref), y_ref, so_ref, qo_ref)
    return body


def _mk_mid_res(count):
    def body(yp_ref, r_ref, w_ref, s_ref, q_ref, g_ref, b_ref,
             y_ref, so_ref, qo_ref):
        scale, shift = _coeffs_in(s_ref, q_ref, g_ref, b_ref, count)
        a = r_ref[...] + _lrelu(yp_ref[...] * scale + shift)
        _emit(_conv3x3_s1(a, w_ref), y_ref, so_ref, qo_ref)
    return body


def _mk_down(count):
    def body(y4_ref, y2_ref, x0_ref, w02_ref, w1_ref,
             s2_ref, q2_ref, g2_ref, b2_ref,
             s4_ref, q4_ref, g4_ref, b4_ref,
             y_ref, so_ref, qo_ref):
        sc2, sh2 = _coeffs_in(s2_ref, q2_ref, g2_ref, b2_ref, count)
        sc4, sh4 = _coeffs_in(s4_ref, q4_ref, g4_ref, b4_ref, count)
        a = (x0_ref[...] + _lrelu(y2_ref[...] * sc2 + sh2)
             + _lrelu(y4_ref[...] * sc4 + sh4))
        b, H, W, C = a.shape
        Ho, Wo = H // 2, W // 2
        cd = y_ref.shape[-1]
        ab = a.astype(jnp.bfloat16)
        ap = jnp.pad(ab, ((0, 0), (1, 1), (1, 1), (0, 0)))
        xc = jnp.concatenate(
            [ap[:, :, 0:W, :], ap[:, :, 1:W + 1, :], ap[:, :, 2:W + 2, :]],
            axis=3)
        hp = (H + 2) // 2
        xs = xc.reshape(b, hp, 2, W, 3 * C)
        xe, xo = xs[:, :, 0], xs[:, :, 1]
        ye = jnp.dot(xe.reshape(b * hp * W, 3 * C), w02_ref[...],
                     preferred_element_type=jnp.float32)
        ye = ye.reshape(b, hp, W, 2 * cd)
        yo = jnp.dot(xo[:, 0:Ho].reshape(b * Ho * W, 3 * C), w1_ref[...],
                     preferred_element_type=jnp.float32)
        acc = (ye[:, 0:Ho, :, 0:cd] + yo.reshape(b, Ho, W, cd)
               + ye[:, 1:Ho + 1, :, cd:2 * cd])
        y = acc.reshape(b, Ho, Wo, 2, cd)[:, :, :, 0, :]
        _emit(y.reshape(b * Ho * Wo, cd), y_ref, so_ref, qo_ref)
    return body


def _mk_bn(count):
    def body(y_ref, s_ref, q_ref, g_ref, b_ref, o_ref):
        scale, shift = _coeffs_in(s_ref, q_ref, g_ref, b_ref, count)
        o_ref[...] = _lrelu(y_ref[...] * scale + shift)
    return body


def _img_spec(h, w, c):
    return pl.BlockSpec((IPB, h, w, c), lambda i: (i, 0, 0, 0))


def _full3_spec(shape):
    return pl.BlockSpec(shape, lambda i: (0, 0, 0))


def _row_spec(c):
    return pl.BlockSpec((1, c), lambda i: (0, 0))


_CP = pltpu.CompilerParams(dimension_semantics=("parallel",),
                           vmem_limit_bytes=100 * 1024 * 1024)


def _stat_in_specs(g, c):
    return [_full3_spec((g, 8, c)), _full3_spec((g, 8, c)),
            _row_spec(c), _row_spec(c)]


def _conv_call(body, ins, in_specs, n, ho, wo, cout, n_extra=0):
    g = n // IPB
    out_shape = ((jax.ShapeDtypeStruct((n, ho, wo, cout), jnp.float32),)
                 * (1 + n_extra)
                 + (jax.ShapeDtypeStruct((g, 8, cout), jnp.float32),
                    jax.ShapeDtypeStruct((g, 8, cout), jnp.float32)))
    out_specs = ((_img_spec(ho, wo, cout),) * (1 + n_extra)
                 + (pl.BlockSpec((1, 8, cout), lambda i: (i, 0, 0)),
                    pl.BlockSpec((1, 8, cout), lambda i: (i, 0, 0))))
    return pl.pallas_call(
        body, out_shape=out_shape, grid=(g,), in_specs=in_specs,
        out_specs=out_specs, compiler_params=_CP)(*ins)


def kernel(x, res_w_0, res_b_0, res_g_0, res_be_0,
           res_w_1, res_b_1, res_g_1, res_be_1,
           down_w, down_b, down_g, down_be):
    n, c, h, w = x.shape
    cd = down_w.shape[0]
    ho, wo = h // 2, w // 2
    g = n // IPB
    m1 = n * h * w
    m5 = n * ho * wo

    x0 = jnp.transpose(x, (0, 2, 3, 1))
    w0 = _prep_w_wide(res_w_0)
    w1 = _prep_w_wide(res_w_1)
    wdp = _prep_w(down_w)
    wd02 = jnp.concatenate([wdp[0], wdp[2]], axis=1)
    wd1 = wdp[1]
    g0, be0 = res_g_0.reshape(1, c), res_be_0.reshape(1, c)
    g1, be1 = res_g_1.reshape(1, c), res_be_1.reshape(1, c)
    gd, bed = down_g.reshape(1, cd), down_be.reshape(1, cd)

    img = _img_spec(h, w, c)
    wsp = pl.BlockSpec((3 * c, 4 * c), lambda i: (0, 0))
    stat = _stat_in_specs(g, c)

    y1, s1, q1 = _conv_call(_k_first, (x0, w0), [img, wsp], n, h, w, c)
    y2, s2, q2 = _conv_call(_mk_mid(m1), (y1, w0, s1, q1, g0, be0),
                            [img, wsp] + stat, n, h, w, c)
    y3, s3, q3 = _conv_call(_mk_mid_res(m1), (y2, x0, w1, s2, q2, g0, be0),
                            [img, img, wsp] + stat, n, h, w, c)
    y4, s4, q4 = _conv_call(_mk_mid(m1), (y3, w1, s3, q3, g1, be1),
                            [img, wsp] + stat, n, h, w, c)
    y5, s5, q5 = _conv_call(
        _mk_down(m1),
        (y4, y2, x0, wd02, wd1, s2, q2, g0, be0, s4, q4, g1, be1),
        [img, img, img,
         pl.BlockSpec((3 * c, 2 * cd), lambda i: (0, 0)),
         pl.BlockSpec((3 * c, cd), lambda i: (0, 0))] + stat + stat,
        n, ho, wo, cd)

    out = pl.pallas_call(
        _mk_bn(m5),
        out_shape=jax.ShapeDtypeStruct((n, ho, wo, cd), jnp.float32),
        grid=(g,),
        in_specs=[_img_spec(ho, wo, cd)] + _stat_in_specs(g, cd),
        out_specs=_img_spec(ho, wo, cd),
        compiler_params=_CP)(y5, s5, q5, gd, bed)

    return jnp.transpose(out, (0, 3, 1, 2))


# speedup vs baseline: 1.0584x; 1.0429x over previous
import jax
import jax.numpy as jnp
from jax.experimental import pallas as pl
from jax.experimental.pallas import tpu as pltpu

LEAKY_SLOPE = 0.01
BN_EPS = 1e-5
IPB = 4


def _prep_w(w_oihw):
    o, i, kh, kw = w_oihw.shape
    w = jnp.transpose(w_oihw, (2, 3, 1, 0)).reshape(kh, kw * i, o)
    return w.astype(jnp.bfloat16)


def _prep_w_wide(w_oihw):
    o, i, kh, kw = w_oihw.shape
    w = jnp.transpose(w_oihw, (2, 3, 1, 0)).reshape(kh, kw * i, o)
    w = jnp.concatenate([w[0], w[1], w[2], jnp.zeros((kw * i, o), w.dtype)],
                        axis=1)
    return w.astype(jnp.bfloat16)


def _lrelu(z):
    return jnp.maximum(z, LEAKY_SLOPE * z)


def _conv3x3_s1(a, w_ref):
    b, H, W, C = a.shape
    co = w_ref.shape[-1] // 4
    ab = a.astype(jnp.bfloat16)
    ap = jnp.pad(ab, ((0, 0), (1, 1), (1, 1), (0, 0)))
    xc = jnp.concatenate(
        [ap[:, :, 0:W, :], ap[:, :, 1:W + 1, :], ap[:, :, 2:W + 2, :]],
        axis=3)
    yw = jnp.dot(xc.reshape(b * (H + 2) * W, 3 * C), w_ref[...],
                 preferred_element_type=jnp.float32)
    yw = yw.reshape(b, H + 2, W, 4 * co)
    y = (yw[:, 0:H, :, 0:co] + yw[:, 1:H + 1, :, co:2 * co]
         + yw[:, 2:H + 2, :, 2 * co:3 * co])
    return y.reshape(b * H * W, co)


def _emit(y, y_ref, s_ref, q_ref):
    y_ref[...] = y.reshape(y_ref.shape).astype(y_ref.dtype)
    yr = y.reshape(-1, 8, y.shape[-1])
    s_ref[0] = jnp.sum(yr, axis=0)
    q_ref[0] = jnp.sum(yr * yr, axis=0)


def _coeffs_in(s_ref, q_ref, g_ref, b_ref, count):
    c = s_ref.shape[-1]
    s = jnp.sum(s_ref[...].reshape(-1, c), axis=0)
    q = jnp.sum(q_ref[...].reshape(-1, c), axis=0)
    mean = s / count
    var = jnp.maximum(q / count - mean * mean, 0.0)
    scale = g_ref[0] * jax.lax.rsqrt(var + BN_EPS)
    shift = b_ref[0] - mean * scale
    return scale, shift


def _k_first(x_ref, w_ref, y_ref, s_ref, q_ref):
    _emit(_conv3x3_s1(x_ref[...], w_ref), y_ref, s_ref, q_ref)


def _mk_mid(count):
    def body(yp_ref, w_ref, s_ref, q_ref, g_ref, b_ref,
             y_ref, so_ref, qo_ref):
        scale, shift = _coeffs_in(s_ref, q_ref, g_ref, b_ref, count)
        a = _lrelu(yp_ref[...] * scale + shift)
        _emit(_conv3x3_s1(a, w_ref), y_ref, so_ref, qo_ref)
    return body


def _mk_mid_res(count):
    def body(yp_ref, r_ref, w_ref, s_ref, q_ref, g_ref, b_ref,
             y_ref, so_ref, qo_ref):
        scale, shift = _coeffs_in(s_ref, q_ref, g_ref, b_ref, count)
        a = r_ref[...] + _lrelu(yp_ref[...] * scale + shift)
        _emit(_conv3x3_s1(a, w_ref), y_ref, so_ref, qo_ref)
    return body


def _mk_down(count):
    def body(y4_ref, y2_ref, x0_ref, w02_ref, w1_ref,
             s2_ref, q2_ref, g2_ref, b2_ref,
             s4_ref, q4_ref, g4_ref, b4_ref,
             y_ref, so_ref, qo_ref):
        sc2, sh2 = _coeffs_in(s2_ref, q2_ref, g2_ref, b2_ref, count)
        sc4, sh4 = _coeffs_in(s4_ref, q4_ref, g4_ref, b4_ref, count)
        a = (x0_ref[...] + _lrelu(y2_ref[...] * sc2 + sh2)
             + _lrelu(y4_ref[...] * sc4 + sh4))
        b, H, W, C = a.shape
        Ho, Wo = H // 2, W // 2
        cd = y_ref.shape[-1]
        ab = a.astype(jnp.bfloat16)
        ap = jnp.pad(ab, ((0, 0), (1, 1), (1, 1), (0, 0)))
        xc = jnp.concatenate(
            [ap[:, :, 0:W, :], ap[:, :, 1:W + 1, :], ap[:, :, 2:W + 2, :]],
            axis=3)
        hp = (H + 2) // 2
        xs = xc.reshape(b, hp, 2, W, 3 * C)
        xe, xo = xs[:, :, 0], xs[:, :, 1]
        ye = jnp.dot(xe.reshape(b * hp * W, 3 * C), w02_ref[...],
                     preferred_element_type=jnp.float32)
        ye = ye.reshape(b, hp, W, 2 * cd)
        yo = jnp.dot(xo[:, 0:Ho].reshape(b * Ho * W, 3 * C), w1_ref[...],
                     preferred_element_type=jnp.float32)
        acc = (ye[:, 0:Ho, :, 0:cd] + yo.reshape(b, Ho, W, cd)
               + ye[:, 1:Ho + 1, :, cd:2 * cd])
        y = acc.reshape(b, Ho, Wo, 2, cd)[:, :, :, 0, :]
        _emit(y.reshape(b * Ho * Wo, cd), y_ref, so_ref, qo_ref)
    return body


def _mk_bn(count):
    def body(y_ref, s_ref, q_ref, g_ref, b_ref, o_ref):
        scale, shift = _coeffs_in(s_ref, q_ref, g_ref, b_ref, count)
        o_ref[...] = _lrelu(y_ref[...] * scale + shift)
    return body


def _img_spec(h, w, c):
    return pl.BlockSpec((IPB, h, w, c), lambda i: (i, 0, 0, 0))


def _full3_spec(shape):
    return pl.BlockSpec(shape, lambda i: (0, 0, 0))


def _row_spec(c):
    return pl.BlockSpec((1, c), lambda i: (0, 0))


_CP = pltpu.CompilerParams(dimension_semantics=("parallel",),
                           vmem_limit_bytes=100 * 1024 * 1024)


def _stat_in_specs(g, c):
    return [_full3_spec((g, 8, c)), _full3_spec((g, 8, c)),
            _row_spec(c), _row_spec(c)]


def _conv_call(body, ins, in_specs, n, ho, wo, cout, n_extra=0):
    g = n // IPB
    out_shape = ((jax.ShapeDtypeStruct((n, ho, wo, cout), jnp.float32),)
                 * (1 + n_extra)
                 + (jax.ShapeDtypeStruct((g, 8, cout), jnp.float32),
                    jax.ShapeDtypeStruct((g, 8, cout), jnp.float32)))
    out_specs = ((_img_spec(ho, wo, cout),) * (1 + n_extra)
                 + (pl.BlockSpec((1, 8, cout), lambda i: (i, 0, 0)),
                    pl.BlockSpec((1, 8, cout), lambda i: (i, 0, 0))))
    return pl.pallas_call(
        body, out_shape=out_shape, grid=(g,), in_specs=in_specs,
        out_specs=out_specs, compiler_params=_CP)(*ins)


def kernel(x, res_w_0, res_b_0, res_g_0, res_be_0,
           res_w_1, res_b_1, res_g_1, res_be_1,
           down_w, down_b, down_g, down_be):
    n, c, h, w = x.shape
    cd = down_w.shape[0]
    ho, wo = h // 2, w // 2
    g = n // IPB
    m1 = n * h * w
    m5 = n * ho * wo

    x0 = jnp.transpose(x, (0, 2, 3, 1))
    w0 = _prep_w_wide(res_w_0)
    w1 = _prep_w_wide(res_w_1)
    wdp = _prep_w(down_w)
    wd02 = jnp.concatenate([wdp[0], wdp[2]], axis=1)
    wd1 = wdp[1]
    g0, be0 = res_g_0.reshape(1, c), res_be_0.reshape(1, c)
    g1, be1 = res_g_1.reshape(1, c), res_be_1.reshape(1, c)
    gd, bed = down_g.reshape(1, cd), down_be.reshape(1, cd)

    img = _img_spec(h, w, c)
    wsp = pl.BlockSpec((3 * c, 4 * c), lambda i: (0, 0))
    stat = _stat_in_specs(g, c)

    y1, s1, q1 = _conv_call(_k_first, (x0, w0), [img, wsp], n, h, w, c)
    y2, s2, q2 = _conv_call(_mk_mid(m1), (y1, w0, s1, q1, g0, be0),
                            [img, wsp] + stat, n, h, w, c)
    y3, s3, q3 = _conv_call(_mk_mid_res(m1), (y2, x0, w1, s2, q2, g0, be0),
                            [img, img, wsp] + stat, n, h, w, c)
    y4, s4, q4 = _conv_call(_mk_mid(m1), (y3, w1, s3, q3, g1, be1),
                            [img, wsp] + stat, n, h, w, c)
    y5, s5, q5 = _conv_call(
        _mk_down(m1),
        (y4, y2, x0, wd02, wd1, s2, q2, g0, be0, s4, q4, g1, be1),
        [img, img, img,
         pl.BlockSpec((3 * c, 2 * cd), lambda i: (0, 0)),
         pl.BlockSpec((3 * c, cd), lambda i: (0, 0))] + stat + stat,
        n, ho, wo, cd)

    out = pl.pallas_call(
        _mk_bn(m5),
        out_shape=jax.ShapeDtypeStruct((n, ho, wo, cd), jnp.float32),
        grid=(g,),
        in_specs=[_img_spec(ho, wo, cd)] + _stat_in_specs(g, cd),
        out_specs=_img_spec(ho, wo, cd),
        compiler_params=_CP)(y5, s5, q5, gd, bed)

    return jnp.transpose(out, (0, 3, 1, 2))
